# baseline (speedup 1.0000x reference)
import functools

import jax
import jax.numpy as jnp
from jax import lax
from jax.experimental import pallas as pl
from jax.experimental.pallas import tpu as pltpu
from jax.experimental.pallas import tpu_sc as plsc

_EMBED_DIM = 64
_D_MODEL = 2 * _EMBED_DIM
_HIDDEN = 2 * _D_MODEL
_STEPS = 4
_BQ = 1024
_BS = 64

_NUM_IDX = _BQ * 2 + _BS * 2
_GATHER_ROWS = 2304
_NC, _NS = 2, 16
_NW = _NC * _NS
_ROWS_PER_W = _GATHER_ROWS // _NW
_PAIR_ROWS = 50000


def _gather_sc(pair_view, idx_half):
    mesh = plsc.VectorSubcoreMesh(core_axis_name="c", subcore_axis_name="s")

    @functools.partial(
        pl.kernel,
        mesh=mesh,
        out_type=jax.ShapeDtypeStruct((_GATHER_ROWS, 2 * _EMBED_DIM),
                                      jnp.float32),
        scratch_types=[
            pltpu.VMEM((_ROWS_PER_W,), jnp.int32),
            pltpu.VMEM((_ROWS_PER_W, 2 * _EMBED_DIM), jnp.float32),
            pltpu.SemaphoreType.DMA,
        ],
    )
    def k(table_hbm, idx_hbm, out_hbm, idx_v, rows_v, sem):
        wid = lax.axis_index("s") * _NC + lax.axis_index("c")
        base = wid * _ROWS_PER_W
        pltpu.sync_copy(idx_hbm.at[pl.ds(base, _ROWS_PER_W)], idx_v)
        pltpu.async_copy(table_hbm.at[idx_v], rows_v, sem).wait()
        pltpu.sync_copy(rows_v, out_hbm.at[pl.ds(base, _ROWS_PER_W)])

    return k(pair_view, idx_half)


def _dot_t(a, b):
    return lax.dot_general(a, b, (((1,), (1,)), ((), ())),
                           preferred_element_type=jnp.float32)


def _pick_half(rows2, p0, p1):
    d = _EMBED_DIM
    a = rows2[:, 0:d] * (1.0 - p0) + rows2[:, d:2 * d] * p0
    b = rows2[:, 2 * d:3 * d] * (1.0 - p1) + rows2[:, 3 * d:4 * d] * p1
    return jnp.concatenate([a, b], axis=1)


def _dense_body(qr_ref, qp0_ref, qp1_ref, sr_ref, sp0_ref, sp1_ref,
                w1_ref, b1_ref, w2_ref, b2_ref, lng_ref, lnb_ref,
                wih_ref, whh_ref, bih_ref, bhh_ref, out_ref):
    q = _pick_half(qr_ref[...], qp0_ref[...], qp1_ref[...])
    s = _pick_half(sr_ref[...], sp0_ref[...], sp1_ref[...])

    hid = jnp.maximum(_dot_t(s, w1_ref[...]) + b1_ref[...], 0.0)
    y = _dot_t(hid, w2_ref[...]) + b2_ref[...] + s
    mu = jnp.mean(y, axis=-1, keepdims=True)
    var = jnp.mean((y - mu) * (y - mu), axis=-1, keepdims=True)
    sg = lng_ref[...] * (y - mu) * lax.rsqrt(var + 1e-5) + lnb_ref[...]

    xg = _dot_t(q, wih_ref[...]) + bih_ref[...] + bhh_ref[...]
    h_r = jnp.zeros((_BQ, _HIDDEN), jnp.float32)
    c = jnp.zeros((_BQ, _HIDDEN), jnp.float32)
    h = q
    for _ in range(_STEPS):
        gates = xg + _dot_t(h_r, whh_ref[...])
        i_g = jax.nn.sigmoid(gates[:, :_HIDDEN])
        f_g = jax.nn.sigmoid(gates[:, _HIDDEN:2 * _HIDDEN])
        g_g = jnp.tanh(gates[:, 2 * _HIDDEN:3 * _HIDDEN])
        o_g = jax.nn.sigmoid(gates[:, 3 * _HIDDEN:])
        c = f_g * c + i_g * g_g
        h_full = o_g * jnp.tanh(c)
        h = q + h_full[:, :_D_MODEL]
        logits = _dot_t(h, sg)
        m = jnp.max(logits, axis=1, keepdims=True)
        e = jnp.exp(logits - m)
        attn = e / jnp.sum(e, axis=1, keepdims=True)
        r = jnp.dot(attn, sg, preferred_element_type=jnp.float32)
        h_r = jnp.concatenate([h, r], axis=1)

    out_ref[...] = _dot_t(h, sg)


def _dense_call(interpret=False):
    return pl.pallas_call(
        _dense_body,
        out_shape=jax.ShapeDtypeStruct((_BQ, _BS), jnp.float32),
        interpret=interpret,
    )


def kernel(query, support, symbol_emb, W1, b1, W2, b2, ln_g, ln_b,
           W_ih, W_hh, b_ih, b_hh):
    idx = jnp.concatenate([
        query.reshape(-1).astype(jnp.int32),
        support.reshape(-1).astype(jnp.int32),
        jnp.zeros((_GATHER_ROWS - _NUM_IDX,), jnp.int32),
    ])
    pair_view = symbol_emb[:2 * _PAIR_ROWS].reshape(_PAIR_ROWS, 2 * _EMBED_DIM)
    rows = _gather_sc(pair_view, idx // 2)
    parity = (idx % 2).astype(jnp.float32)

    qr = rows[:_BQ * 2].reshape(_BQ, 4 * _EMBED_DIM)
    qp0 = parity[:_BQ * 2:2].reshape(_BQ, 1)
    qp1 = parity[1:_BQ * 2:2].reshape(_BQ, 1)
    sr = rows[_BQ * 2:_NUM_IDX].reshape(_BS, 4 * _EMBED_DIM)
    sp0 = parity[_BQ * 2:_NUM_IDX:2].reshape(_BS, 1)
    sp1 = parity[_BQ * 2 + 1:_NUM_IDX:2].reshape(_BS, 1)

    return _dense_call()(
        qr, qp0, qp1, sr, sp0, sp1,
        W1, b1.reshape(1, -1), W2, b2.reshape(1, -1),
        ln_g.reshape(1, -1), ln_b.reshape(1, -1), W_ih, W_hh,
        b_ih.reshape(1, -1), b_hh.reshape(1, -1))


# speedup vs baseline: 1.0564x; 1.0564x over previous
import functools

import jax
import jax.numpy as jnp
from jax import lax
from jax.experimental import pallas as pl
from jax.experimental.pallas import tpu as pltpu
from jax.experimental.pallas import tpu_sc as plsc

_EMBED_DIM = 64
_D_MODEL = 2 * _EMBED_DIM
_HIDDEN = 2 * _D_MODEL
_STEPS = 4
_BQ = 1024
_BS = 64

_NUM_IDX = _BQ * 2 + _BS * 2
_GATHER_ROWS = 2304
_NC, _NS = 2, 16
_NW = _NC * _NS
_ROWS_PER_W = _GATHER_ROWS // _NW
_PAIR_ROWS = 50000


def _gather_sc(table, idx):
    mesh = plsc.VectorSubcoreMesh(core_axis_name="c", subcore_axis_name="s")

    @functools.partial(
        pl.kernel,
        mesh=mesh,
        out_type=jax.ShapeDtypeStruct((_GATHER_ROWS, _EMBED_DIM),
                                      jnp.float32),
        scratch_types=[
            pltpu.VMEM((_ROWS_PER_W,), jnp.int32),
            pltpu.VMEM((_ROWS_PER_W, _EMBED_DIM), jnp.float32),
            pltpu.SemaphoreType.DMA,
        ],
        compiler_params=pltpu.CompilerParams(use_tc_tiling_on_sc=False),
    )
    def k(table_hbm, idx_hbm, out_hbm, idx_v, rows_v, sem):
        wid = lax.axis_index("s") * _NC + lax.axis_index("c")
        base = wid * _ROWS_PER_W
        pltpu.sync_copy(idx_hbm.at[pl.ds(base, _ROWS_PER_W)], idx_v)
        pltpu.async_copy(table_hbm.at[idx_v], rows_v, sem).wait()
        pltpu.sync_copy(rows_v, out_hbm.at[pl.ds(base, _ROWS_PER_W)])

    return k(table, idx)


def _dot_t(a, b):
    return lax.dot_general(a, b, (((1,), (1,)), ((), ())),
                           preferred_element_type=jnp.float32)


def _dense_body(q_ref, s_ref,
                w1_ref, b1_ref, w2_ref, b2_ref, lng_ref, lnb_ref,
                wih_ref, whh_ref, bih_ref, bhh_ref, out_ref):
    q = q_ref[...]
    s = s_ref[...]

    hid = jnp.maximum(_dot_t(s, w1_ref[...]) + b1_ref[...], 0.0)
    y = _dot_t(hid, w2_ref[...]) + b2_ref[...] + s
    mu = jnp.mean(y, axis=-1, keepdims=True)
    var = jnp.mean((y - mu) * (y - mu), axis=-1, keepdims=True)
    sg = lng_ref[...] * (y - mu) * lax.rsqrt(var + 1e-5) + lnb_ref[...]

    xg = _dot_t(q, wih_ref[...]) + bih_ref[...] + bhh_ref[...]
    h_r = jnp.zeros((_BQ, _HIDDEN), jnp.float32)
    c = jnp.zeros((_BQ, _HIDDEN), jnp.float32)
    h = q
    for _ in range(_STEPS):
        gates = xg + _dot_t(h_r, whh_ref[...])
        i_g = jax.nn.sigmoid(gates[:, :_HIDDEN])
        f_g = jax.nn.sigmoid(gates[:, _HIDDEN:2 * _HIDDEN])
        g_g = jnp.tanh(gates[:, 2 * _HIDDEN:3 * _HIDDEN])
        o_g = jax.nn.sigmoid(gates[:, 3 * _HIDDEN:])
        c = f_g * c + i_g * g_g
        h_full = o_g * jnp.tanh(c)
        h = q + h_full[:, :_D_MODEL]
        logits = _dot_t(h, sg)
        m = jnp.max(logits, axis=1, keepdims=True)
        e = jnp.exp(logits - m)
        attn = e / jnp.sum(e, axis=1, keepdims=True)
        r = jnp.dot(attn, sg, preferred_element_type=jnp.float32)
        h_r = jnp.concatenate([h, r], axis=1)

    out_ref[...] = _dot_t(h, sg)


def _dense_call(interpret=False):
    return pl.pallas_call(
        _dense_body,
        out_shape=jax.ShapeDtypeStruct((_BQ, _BS), jnp.float32),
        interpret=interpret,
    )


def kernel(query, support, symbol_emb, W1, b1, W2, b2, ln_g, ln_b,
           W_ih, W_hh, b_ih, b_hh):
    idx = jnp.concatenate([
        query.reshape(-1).astype(jnp.int32),
        support.reshape(-1).astype(jnp.int32),
        jnp.zeros((_GATHER_ROWS - _NUM_IDX,), jnp.int32),
    ])
    rows = _gather_sc(symbol_emb, idx)
    q = rows[:_BQ * 2].reshape(_BQ, _D_MODEL)
    s = rows[_BQ * 2:_NUM_IDX].reshape(_BS, _D_MODEL)

    return _dense_call()(
        q, s,
        W1, b1.reshape(1, -1), W2, b2.reshape(1, -1),
        ln_g.reshape(1, -1), ln_b.reshape(1, -1), W_ih, W_hh,
        b_ih.reshape(1, -1), b_hh.reshape(1, -1))
